# baseline (speedup 1.0000x reference)
import jax
import jax.numpy as jnp
from jax.experimental import pallas as pl

N = 10000
E = 320000
D_NODE = 128
D_EDGE = 16
DIM = 16
EHID = 32
NG = 64
PSTEPS = 3


def _lin0_body(x_ref, w_ref, b_ref, o_ref):
    o_ref[...] = jax.nn.relu(
        jnp.dot(x_ref[...], w_ref[...], preferred_element_type=jnp.float32)
        + b_ref[...]
    )


def _lin0(x, W0, b0):
    TN = 1000
    return pl.pallas_call(
        _lin0_body,
        grid=(N // TN,),
        in_specs=[
            pl.BlockSpec((TN, D_NODE), lambda i: (i, 0)),
            pl.BlockSpec((D_NODE, DIM), lambda i: (0, 0)),
            pl.BlockSpec((DIM,), lambda i: (0,)),
        ],
        out_specs=pl.BlockSpec((TN, DIM), lambda i: (i, 0)),
        out_shape=jax.ShapeDtypeStruct((N, DIM), jnp.float32),
    )(x, W0, b0)


def kernel(x, edge_index, edge_attr, batch, W0, b0, We1, be1, We2, be2,
           Wroot, conv_b, gru_Wih, gru_Whh, gru_bih, gru_bhh,
           lstm_Wih, lstm_Whh, lstm_bih, lstm_bhh, W1, b1, W2, b2):
    src = edge_index[0]
    dst = edge_index[1]
    out = _lin0(x, W0, b0)
    h = out
    ew = (jax.nn.relu(edge_attr @ We1 + be1) @ We2 + be2).reshape(-1, DIM, DIM)
    ones = jnp.ones((E,), jnp.float32)
    counts = jnp.clip(jax.ops.segment_sum(ones, dst, num_segments=N), 1.0, None)
    for _ in range(3):
        msg = jnp.einsum('ei,eio->eo', out[src], ew)
        aggr = jax.ops.segment_sum(msg, dst, num_segments=N) / counts[:, None]
        m = jax.nn.relu(aggr + out @ Wroot + conv_b)
        gi = m @ gru_Wih.T + gru_bih
        gh = h @ gru_Whh.T + gru_bhh
        i_r, i_z, i_n = jnp.split(gi, 3, axis=-1)
        h_r, h_z, h_n = jnp.split(gh, 3, axis=-1)
        r = jax.nn.sigmoid(i_r + h_r)
        z = jax.nn.sigmoid(i_z + h_z)
        n = jnp.tanh(i_n + r * h_n)
        h = (1.0 - z) * n + z * h
        out = h
    q_star = jnp.zeros((NG, 2 * DIM), jnp.float32)
    hs = jnp.zeros((NG, DIM), jnp.float32)
    cs = jnp.zeros((NG, DIM), jnp.float32)
    for _ in range(PSTEPS):
        gi = q_star @ lstm_Wih.T + lstm_bih
        gh = hs @ lstm_Whh.T + lstm_bhh
        a_i, a_f, a_g, a_o = jnp.split(gi + gh, 4, axis=-1)
        ig = jax.nn.sigmoid(a_i)
        fg = jax.nn.sigmoid(a_f)
        gg = jnp.tanh(a_g)
        og = jax.nn.sigmoid(a_o)
        cs = fg * cs + ig * gg
        hs = og * jnp.tanh(cs)
        q = hs
        e = jnp.sum(out * q[batch], axis=-1)
        emax = jax.ops.segment_max(e, batch, num_segments=NG)
        a = jnp.exp(e - emax[batch])
        asum = jax.ops.segment_sum(a, batch, num_segments=NG) + 1e-16
        alpha = a / asum[batch]
        rvec = jax.ops.segment_sum(alpha[:, None] * out, batch, num_segments=NG)
        q_star = jnp.concatenate([q, rvec], axis=-1)
    o = jax.nn.relu(q_star @ W1 + b1)
    o = o @ W2 + b2
    return o.reshape(-1)


# speedup vs baseline: 1.4582x; 1.4582x over previous
import functools

import jax
import jax.numpy as jnp
from jax import lax
from jax.experimental import pallas as pl
from jax.experimental.pallas import tpu as pltpu

N = 10000
E = 320000
D_NODE = 128
D_EDGE = 16
DIM = 16
EHID = 32
NG = 64
PSTEPS = 3

TEL = 2560


def _lin0_body(x_ref, w_ref, b_ref, o_ref):
    o_ref[...] = jax.nn.relu(
        jnp.dot(x_ref[...], w_ref[...], preferred_element_type=jnp.float32)
        + b_ref[...]
    )


def _lin0(x, W0, b0):
    return pl.pallas_call(
        _lin0_body,
        grid=(5,),
        in_specs=[
            pl.BlockSpec((N // 5, D_NODE), lambda i: (i, 0)),
            pl.BlockSpec((D_NODE, DIM), lambda i: (0, 0)),
            pl.BlockSpec((1, DIM), lambda i: (0, 0)),
        ],
        out_specs=pl.BlockSpec((N // 5, DIM), lambda i: (i, 0)),
        out_shape=jax.ShapeDtypeStruct((N, DIM), jnp.float32),
    )(x, W0, b0.reshape(1, DIM))


def _ewt_body(ea_ref, w1_ref, b1_ref, w2_ref, b2_ref, o_ref):
    a = jax.nn.relu(
        jnp.dot(w1_ref[...], ea_ref[...], preferred_element_type=jnp.float32)
        + b1_ref[...]
    )
    o_ref[...] = (
        jnp.dot(w2_ref[...], a, preferred_element_type=jnp.float32) + b2_ref[...]
    )


def _ewt(ea_t, We1, be1, We2, be2):
    return pl.pallas_call(
        _ewt_body,
        grid=(E // TEL,),
        in_specs=[
            pl.BlockSpec((D_EDGE, TEL), lambda i: (0, i)),
            pl.BlockSpec((EHID, D_EDGE), lambda i: (0, 0)),
            pl.BlockSpec((EHID, 1), lambda i: (0, 0)),
            pl.BlockSpec((DIM * DIM, EHID), lambda i: (0, 0)),
            pl.BlockSpec((DIM * DIM, 1), lambda i: (0, 0)),
        ],
        out_specs=pl.BlockSpec((DIM * DIM, TEL), lambda i: (0, i)),
        out_shape=jax.ShapeDtypeStruct((DIM * DIM, E), jnp.float32),
    )(ea_t, We1.T, be1.reshape(EHID, 1), We2.T, be2.reshape(DIM * DIM, 1))


def _msg_body(x_ref, ew_ref, o_ref, *, width):
    x = x_ref[...]
    acc = jnp.zeros((DIM, x.shape[1]), jnp.float32)
    for i in range(DIM):
        acc = acc + x[i : i + 1, :] * ew_ref[pl.ds(i * DIM, DIM), :]
    if width == DIM:
        o_ref[...] = acc
    else:
        o_ref[...] = jnp.concatenate(
            [acc, jnp.ones((width - DIM, x.shape[1]), jnp.float32)], axis=0
        )


def _msg(x_src_t, ew_t, width):
    return pl.pallas_call(
        functools.partial(_msg_body, width=width),
        grid=(E // TEL,),
        in_specs=[
            pl.BlockSpec((DIM, TEL), lambda i: (0, i)),
            pl.BlockSpec((DIM * DIM, TEL), lambda i: (0, i)),
        ],
        out_specs=pl.BlockSpec((width, TEL), lambda i: (0, i)),
        out_shape=jax.ShapeDtypeStruct((width, E), jnp.float32),
    )(x_src_t, ew_t)


def _node_body(a0_ref, a1_ref, cr_ref, out_ref, wroot_ref, cb_ref,
               wih_ref, whh_ref, bih_ref, bhh_ref, o_ref):
    aggr = (a0_ref[:, :DIM] + a1_ref[:, :DIM]) * cr_ref[...]
    out = out_ref[...]
    m = jax.nn.relu(
        aggr
        + jnp.dot(out, wroot_ref[...], preferred_element_type=jnp.float32)
        + cb_ref[...]
    )
    gi = jnp.dot(m, wih_ref[...], preferred_element_type=jnp.float32) + bih_ref[...]
    gh = jnp.dot(out, whh_ref[...], preferred_element_type=jnp.float32) + bhh_ref[...]
    r = jax.nn.sigmoid(gi[:, :DIM] + gh[:, :DIM])
    z = jax.nn.sigmoid(gi[:, DIM : 2 * DIM] + gh[:, DIM : 2 * DIM])
    n = jnp.tanh(gi[:, 2 * DIM :] + r * gh[:, 2 * DIM :])
    o_ref[...] = (1.0 - z) * n + z * out


def _counts_body(a0_ref, a1_ref, o_ref):
    c = a0_ref[:, DIM:] + a1_ref[:, DIM:]
    o_ref[...] = 1.0 / jnp.clip(c, 1.0, None)


def _node(acc, countsr, out, Wroot, conv_b, gru_Wih, gru_Whh, gru_bih, gru_bhh):
    w = acc.shape[-1]
    return pl.pallas_call(
        _node_body,
        grid=(1,),
        in_specs=[
            pl.BlockSpec((N, w), lambda i: (0, 0)),
            pl.BlockSpec((N, w), lambda i: (0, 0)),
            pl.BlockSpec((N, DIM), lambda i: (0, 0)),
            pl.BlockSpec((N, DIM), lambda i: (0, 0)),
            pl.BlockSpec((DIM, DIM), lambda i: (0, 0)),
            pl.BlockSpec((1, DIM), lambda i: (0, 0)),
            pl.BlockSpec((DIM, 3 * DIM), lambda i: (0, 0)),
            pl.BlockSpec((DIM, 3 * DIM), lambda i: (0, 0)),
            pl.BlockSpec((1, 3 * DIM), lambda i: (0, 0)),
            pl.BlockSpec((1, 3 * DIM), lambda i: (0, 0)),
        ],
        out_specs=pl.BlockSpec((N, DIM), lambda i: (0, 0)),
        out_shape=jax.ShapeDtypeStruct((N, DIM), jnp.float32),
    )(acc[0], acc[1], countsr, out, Wroot, conv_b.reshape(1, DIM),
      gru_Wih.T, gru_Whh.T, gru_bih.reshape(1, 3 * DIM),
      gru_bhh.reshape(1, 3 * DIM))


def _countsr(acc):
    w = acc.shape[-1]
    return pl.pallas_call(
        _counts_body,
        grid=(1,),
        in_specs=[
            pl.BlockSpec((N, w), lambda i: (0, 0)),
            pl.BlockSpec((N, w), lambda i: (0, 0)),
        ],
        out_specs=pl.BlockSpec((N, w - DIM), lambda i: (0, 0)),
        out_shape=jax.ShapeDtypeStruct((N, w - DIM), jnp.float32),
    )(acc[0], acc[1])


def _s2s_body(out_ref, b_ref, bt_ref, wih_ref, whh_ref, bih_ref, bhh_ref,
              w1_ref, b1_ref, w2_ref, b2_ref, o_ref):
    out = out_ref[...]
    bcol = b_ref[...]
    brow = bt_ref[...]
    gids = lax.broadcasted_iota(jnp.int32, (1, NG), 1)
    gidc = lax.broadcasted_iota(jnp.int32, (NG, 1), 0)
    M = jnp.where(bcol == gids, 1.0, 0.0)
    MT = jnp.where(gidc == brow, 1.0, 0.0)

    q_star = jnp.zeros((NG, 2 * DIM), jnp.float32)
    hs = jnp.zeros((NG, DIM), jnp.float32)
    cs = jnp.zeros((NG, DIM), jnp.float32)
    for _ in range(PSTEPS):
        gi = jnp.dot(q_star, wih_ref[...], preferred_element_type=jnp.float32) + bih_ref[...]
        gh = jnp.dot(hs, whh_ref[...], preferred_element_type=jnp.float32) + bhh_ref[...]
        g = gi + gh
        ig = jax.nn.sigmoid(g[:, :DIM])
        fg = jax.nn.sigmoid(g[:, DIM : 2 * DIM])
        gg = jnp.tanh(g[:, 2 * DIM : 3 * DIM])
        og = jax.nn.sigmoid(g[:, 3 * DIM :])
        cs = fg * cs + ig * gg
        hs = og * jnp.tanh(cs)
        qb = jnp.dot(M, hs, preferred_element_type=jnp.float32)
        e = jnp.sum(out * qb, axis=1, keepdims=True)
        emax = jnp.max(jnp.where(M > 0.0, e, -jnp.inf), axis=0, keepdims=True)
        a = jnp.exp(e - jnp.dot(M, emax.T, preferred_element_type=jnp.float32))
        asum = jnp.dot(MT, a, preferred_element_type=jnp.float32) + 1e-16
        alpha = a / jnp.dot(M, asum, preferred_element_type=jnp.float32)
        rvec = jnp.dot(MT, alpha * out, preferred_element_type=jnp.float32)
        q_star = jnp.concatenate([hs, rvec], axis=1)
    o = jax.nn.relu(
        jnp.dot(q_star, w1_ref[...], preferred_element_type=jnp.float32) + b1_ref[...]
    )
    o_ref[...] = jnp.dot(o, w2_ref[...], preferred_element_type=jnp.float32) + b2_ref[...]


def _s2s(out, batch, lstm_Wih, lstm_Whh, lstm_bih, lstm_bhh, W1, b1, W2, b2):
    return pl.pallas_call(
        _s2s_body,
        grid=(1,),
        in_specs=[
            pl.BlockSpec((N, DIM), lambda i: (0, 0)),
            pl.BlockSpec((N, 1), lambda i: (0, 0)),
            pl.BlockSpec((1, N), lambda i: (0, 0)),
            pl.BlockSpec((2 * DIM, 4 * DIM), lambda i: (0, 0)),
            pl.BlockSpec((DIM, 4 * DIM), lambda i: (0, 0)),
            pl.BlockSpec((1, 4 * DIM), lambda i: (0, 0)),
            pl.BlockSpec((1, 4 * DIM), lambda i: (0, 0)),
            pl.BlockSpec((2 * DIM, DIM), lambda i: (0, 0)),
            pl.BlockSpec((1, DIM), lambda i: (0, 0)),
            pl.BlockSpec((DIM, 1), lambda i: (0, 0)),
            pl.BlockSpec((1, 1), lambda i: (0, 0)),
        ],
        out_specs=pl.BlockSpec((NG, 1), lambda i: (0, 0)),
        out_shape=jax.ShapeDtypeStruct((NG, 1), jnp.float32),
    )(out, batch.reshape(N, 1), batch.reshape(1, N),
      lstm_Wih.T, lstm_Whh.T, lstm_bih.reshape(1, 4 * DIM),
      lstm_bhh.reshape(1, 4 * DIM), W1, b1.reshape(1, DIM), W2,
      b2.reshape(1, 1))


def kernel(x, edge_index, edge_attr, batch, W0, b0, We1, be1, We2, be2,
           Wroot, conv_b, gru_Wih, gru_Whh, gru_bih, gru_bhh,
           lstm_Wih, lstm_Whh, lstm_bih, lstm_bhh, W1, b1, W2, b2):
    src = edge_index[0]
    dst = edge_index[1]
    out = _lin0(x, W0, b0)
    ew_t = _ewt(edge_attr.T, We1, be1, We2, be2)

    countsr = None
    for it in range(3):
        width = 2 * DIM if it == 0 else DIM
        x_src_t = out[src].T
        msg_t = _msg(x_src_t, ew_t, width)
        part = jax.ops.segment_sum(msg_t.T, dst, num_segments=N)
        acc = jnp.stack([part, jnp.zeros_like(part)])
        if it == 0:
            countsr = _countsr(acc)
        out = _node(acc, countsr, out, Wroot, conv_b,
                    gru_Wih, gru_Whh, gru_bih, gru_bhh)

    o = _s2s(out, batch, lstm_Wih, lstm_Whh, lstm_bih, lstm_bhh, W1, b1, W2, b2)
    return o.reshape(-1)


# speedup vs baseline: 4.2133x; 2.8894x over previous
import functools

import jax
import jax.numpy as jnp
from jax import lax
from jax.experimental import pallas as pl
from jax.experimental.pallas import tpu as pltpu
from jax.experimental.pallas import tpu_sc as plsc

N = 10000
E = 320000
D_NODE = 128
D_EDGE = 16
DIM = 16
EHID = 32
NG = 64
PSTEPS = 3

TEL = 2560


def _bdot(a, b):
    return jnp.dot(a.astype(jnp.bfloat16), b.astype(jnp.bfloat16),
                   preferred_element_type=jnp.float32)

NC = 2
NS = 16
NW = NC * NS
EPW = E // NW
CH = 80
REG = 25
REGROWS = REG * CH
NREG = EPW // REGROWS

_sc_mesh = plsc.VectorSubcoreMesh(
    core_axis_name="c", subcore_axis_name="s", num_cores=NC, num_subcores=NS
)


def _gather_sc(table, src):
    @functools.partial(
        pl.kernel,
        out_type=jax.ShapeDtypeStruct((E, DIM), jnp.float32),
        mesh=_sc_mesh,
        compiler_params=pltpu.CompilerParams(use_tc_tiling_on_sc=False),
        scratch_types=[
            pltpu.VMEM((EPW,), jnp.int32),
            pltpu.VMEM((REGROWS, DIM), jnp.float32),
            pltpu.SemaphoreType.DMA,
        ],
    )
    def k(table_hbm, src_hbm, out_hbm, idx_v, buf_v, sem):
        wid = lax.axis_index("s") * NC + lax.axis_index("c")
        base = wid * EPW
        pltpu.sync_copy(src_hbm.at[pl.ds(base, EPW)], idx_v)

        def region(r, carry):
            rbase = r * REGROWS
            ds = []
            for j in range(REG):
                ds.append(pltpu.async_copy(
                    table_hbm.at[idx_v.at[pl.ds(rbase + j * CH, CH)]],
                    buf_v.at[pl.ds(j * CH, CH)],
                    sem,
                ))
            for d in ds:
                d.wait()
            pltpu.sync_copy(buf_v, out_hbm.at[pl.ds(base + rbase, REGROWS)])
            return carry

        lax.fori_loop(0, NREG, region, 0)

    return k(table, src)


def _scatter_sc(msg, dst3, zeros, width):
    @functools.partial(
        pl.kernel,
        out_type=jax.ShapeDtypeStruct((NC, N, width), jnp.float32),
        mesh=_sc_mesh,
        compiler_params=pltpu.CompilerParams(use_tc_tiling_on_sc=False),
        scratch_types=[
            pltpu.VMEM((EPW // CH, CH), jnp.int32),
            pltpu.VMEM((REGROWS, width), jnp.float32),
            pltpu.VMEM_SHARED((N, width), jnp.float32),
            pltpu.SemaphoreType.DMA,
        ],
    )
    def k(msg_hbm, dst_hbm, z_hbm, out_hbm, idx_v, buf_v, acc_sh, sem):
        cid = lax.axis_index("c")
        sid = lax.axis_index("s")
        wid = sid * NC + cid
        base = wid * EPW

        @pl.when(sid == 0)
        def _():
            pltpu.sync_copy(z_hbm, acc_sh)

        pltpu.sync_copy(dst_hbm.at[wid], idx_v)
        plsc.subcore_barrier()

        def region(r, carry):
            pltpu.sync_copy(msg_hbm.at[pl.ds(base + r * REGROWS, REGROWS)], buf_v)
            ds = []
            for j in range(REG):
                ds.append(pltpu.async_copy(
                    buf_v.at[pl.ds(j * CH, CH)],
                    acc_sh.at[idx_v.at[r * REG + j]],
                    sem,
                    add=True,
                ))
            for d in ds:
                d.wait()
            return carry

        lax.fori_loop(0, NREG, region, 0)
        plsc.subcore_barrier()

        @pl.when(sid == 0)
        def _():
            pltpu.sync_copy(acc_sh, out_hbm.at[cid])

    return k(msg, dst3, zeros)


def _lin0_body(x_ref, w_ref, b_ref, o_ref):
    o_ref[...] = jax.nn.relu(_bdot(x_ref[...], w_ref[...]) + b_ref[...])


def _lin0(x, W0, b0):
    return pl.pallas_call(
        _lin0_body,
        grid=(5,),
        in_specs=[
            pl.BlockSpec((N // 5, D_NODE), lambda i: (i, 0)),
            pl.BlockSpec((D_NODE, DIM), lambda i: (0, 0)),
            pl.BlockSpec((1, DIM), lambda i: (0, 0)),
        ],
        out_specs=pl.BlockSpec((N // 5, DIM), lambda i: (i, 0)),
        out_shape=jax.ShapeDtypeStruct((N, DIM), jnp.float32),
    )(x, W0, b0.reshape(1, DIM))


def _ewt_body(ea_ref, w1_ref, b1_ref, w2_ref, b2_ref, o_ref):
    a = jax.nn.relu(_bdot(w1_ref[...], ea_ref[...]) + b1_ref[...])
    o_ref[...] = (_bdot(w2_ref[...], a) + b2_ref[...]).astype(jnp.bfloat16)


def _ewt(ea_t, We1, be1, We2, be2):
    return pl.pallas_call(
        _ewt_body,
        grid=(E // TEL,),
        in_specs=[
            pl.BlockSpec((D_EDGE, TEL), lambda i: (0, i)),
            pl.BlockSpec((EHID, D_EDGE), lambda i: (0, 0)),
            pl.BlockSpec((EHID, 1), lambda i: (0, 0)),
            pl.BlockSpec((DIM * DIM, EHID), lambda i: (0, 0)),
            pl.BlockSpec((DIM * DIM, 1), lambda i: (0, 0)),
        ],
        out_specs=pl.BlockSpec((DIM * DIM, TEL), lambda i: (0, i)),
        out_shape=jax.ShapeDtypeStruct((DIM * DIM, E), jnp.bfloat16),
    )(ea_t, We1.T, be1.reshape(EHID, 1), We2.T, be2.reshape(DIM * DIM, 1))


def _msg_body(x_ref, ew_ref, o_ref, *, width):
    x = x_ref[...].T.astype(jnp.bfloat16).astype(jnp.float32)
    acc = jnp.zeros((DIM, x.shape[1]), jnp.float32)
    for i in range(DIM):
        acc = acc + x[i : i + 1, :] * ew_ref[pl.ds(i * DIM, DIM), :].astype(jnp.float32)
    if width == DIM:
        o_ref[...] = acc.T
    else:
        o_ref[...] = jnp.concatenate(
            [acc.T, jnp.ones((acc.shape[1], width - DIM), jnp.float32)], axis=1
        )


def _msg(x_src, ew_t, width):
    return pl.pallas_call(
        functools.partial(_msg_body, width=width),
        grid=(E // TEL,),
        in_specs=[
            pl.BlockSpec((TEL, DIM), lambda i: (i, 0)),
            pl.BlockSpec((DIM * DIM, TEL), lambda i: (0, i)),
        ],
        out_specs=pl.BlockSpec((TEL, width), lambda i: (i, 0)),
        out_shape=jax.ShapeDtypeStruct((E, width), jnp.float32),
    )(x_src, ew_t)


def _node_body(a0_ref, a1_ref, cr_ref, out_ref, wroot_ref, cb_ref,
               wih_ref, whh_ref, bih_ref, bhh_ref, o_ref):
    aggr = (a0_ref[:, :DIM] + a1_ref[:, :DIM]) * cr_ref[...]
    out = out_ref[...]
    m = jax.nn.relu(
        aggr
        + _bdot(out, wroot_ref[...])
        + cb_ref[...]
    )
    gi = _bdot(m, wih_ref[...]) + bih_ref[...]
    gh = _bdot(out, whh_ref[...]) + bhh_ref[...]
    r = jax.nn.sigmoid(gi[:, :DIM] + gh[:, :DIM])
    z = jax.nn.sigmoid(gi[:, DIM : 2 * DIM] + gh[:, DIM : 2 * DIM])
    n = jnp.tanh(gi[:, 2 * DIM :] + r * gh[:, 2 * DIM :])
    o_ref[...] = (1.0 - z) * n + z * out


def _counts_body(a0_ref, a1_ref, o_ref):
    c = a0_ref[:, DIM:] + a1_ref[:, DIM:]
    o_ref[...] = 1.0 / jnp.clip(c, 1.0, None)


def _node(acc, countsr, out, Wroot, conv_b, gru_Wih, gru_Whh, gru_bih, gru_bhh):
    w = acc.shape[-1]
    return pl.pallas_call(
        _node_body,
        grid=(1,),
        in_specs=[
            pl.BlockSpec((N, w), lambda i: (0, 0)),
            pl.BlockSpec((N, w), lambda i: (0, 0)),
            pl.BlockSpec((N, DIM), lambda i: (0, 0)),
            pl.BlockSpec((N, DIM), lambda i: (0, 0)),
            pl.BlockSpec((DIM, DIM), lambda i: (0, 0)),
            pl.BlockSpec((1, DIM), lambda i: (0, 0)),
            pl.BlockSpec((DIM, 3 * DIM), lambda i: (0, 0)),
            pl.BlockSpec((DIM, 3 * DIM), lambda i: (0, 0)),
            pl.BlockSpec((1, 3 * DIM), lambda i: (0, 0)),
            pl.BlockSpec((1, 3 * DIM), lambda i: (0, 0)),
        ],
        out_specs=pl.BlockSpec((N, DIM), lambda i: (0, 0)),
        out_shape=jax.ShapeDtypeStruct((N, DIM), jnp.float32),
    )(acc[0], acc[1], countsr, out, Wroot, conv_b.reshape(1, DIM),
      gru_Wih.T, gru_Whh.T, gru_bih.reshape(1, 3 * DIM),
      gru_bhh.reshape(1, 3 * DIM))


def _countsr(acc):
    w = acc.shape[-1]
    return pl.pallas_call(
        _counts_body,
        grid=(1,),
        in_specs=[
            pl.BlockSpec((N, w), lambda i: (0, 0)),
            pl.BlockSpec((N, w), lambda i: (0, 0)),
        ],
        out_specs=pl.BlockSpec((N, w - DIM), lambda i: (0, 0)),
        out_shape=jax.ShapeDtypeStruct((N, w - DIM), jnp.float32),
    )(acc[0], acc[1])


def _s2s_body(out_ref, b_ref, bt_ref, wih_ref, whh_ref, bih_ref, bhh_ref,
              w1_ref, b1_ref, w2_ref, b2_ref, o_ref):
    out = out_ref[...]
    bcol = b_ref[...]
    brow = bt_ref[...]
    gids = lax.broadcasted_iota(jnp.int32, (1, NG), 1)
    gidc = lax.broadcasted_iota(jnp.int32, (NG, 1), 0)
    M = jnp.where(bcol == gids, 1.0, 0.0)
    MT = jnp.where(gidc == brow, 1.0, 0.0)

    q_star = jnp.zeros((NG, 2 * DIM), jnp.float32)
    hs = jnp.zeros((NG, DIM), jnp.float32)
    cs = jnp.zeros((NG, DIM), jnp.float32)
    for _ in range(PSTEPS):
        gi = _bdot(q_star, wih_ref[...]) + bih_ref[...]
        gh = _bdot(hs, whh_ref[...]) + bhh_ref[...]
        g = gi + gh
        ig = jax.nn.sigmoid(g[:, :DIM])
        fg = jax.nn.sigmoid(g[:, DIM : 2 * DIM])
        gg = jnp.tanh(g[:, 2 * DIM : 3 * DIM])
        og = jax.nn.sigmoid(g[:, 3 * DIM :])
        cs = fg * cs + ig * gg
        hs = og * jnp.tanh(cs)
        qb = jnp.dot(M, hs, preferred_element_type=jnp.float32, precision=jax.lax.Precision.HIGHEST)
        e = jnp.sum(out * qb, axis=1, keepdims=True)
        emax = jnp.max(jnp.where(M > 0.0, e, -jnp.inf), axis=0, keepdims=True)
        a = jnp.exp(e - jnp.dot(M, emax.T, preferred_element_type=jnp.float32, precision=jax.lax.Precision.HIGHEST))
        asum = jnp.dot(MT, a, preferred_element_type=jnp.float32, precision=jax.lax.Precision.HIGHEST) + 1e-16
        alpha = a / jnp.dot(M, asum, preferred_element_type=jnp.float32, precision=jax.lax.Precision.HIGHEST)
        rvec = jnp.dot(MT, alpha * out, preferred_element_type=jnp.float32, precision=jax.lax.Precision.HIGHEST)
        q_star = jnp.concatenate([hs, rvec], axis=1)
    o = jax.nn.relu(
        _bdot(q_star, w1_ref[...]) + b1_ref[...]
    )
    o_ref[...] = _bdot(o, w2_ref[...]) + b2_ref[...]


def _s2s(out, batch, lstm_Wih, lstm_Whh, lstm_bih, lstm_bhh, W1, b1, W2, b2):
    return pl.pallas_call(
        _s2s_body,
        grid=(1,),
        in_specs=[
            pl.BlockSpec((N, DIM), lambda i: (0, 0)),
            pl.BlockSpec((N, 1), lambda i: (0, 0)),
            pl.BlockSpec((1, N), lambda i: (0, 0)),
            pl.BlockSpec((2 * DIM, 4 * DIM), lambda i: (0, 0)),
            pl.BlockSpec((DIM, 4 * DIM), lambda i: (0, 0)),
            pl.BlockSpec((1, 4 * DIM), lambda i: (0, 0)),
            pl.BlockSpec((1, 4 * DIM), lambda i: (0, 0)),
            pl.BlockSpec((2 * DIM, DIM), lambda i: (0, 0)),
            pl.BlockSpec((1, DIM), lambda i: (0, 0)),
            pl.BlockSpec((DIM, 1), lambda i: (0, 0)),
            pl.BlockSpec((1, 1), lambda i: (0, 0)),
        ],
        out_specs=pl.BlockSpec((NG, 1), lambda i: (0, 0)),
        out_shape=jax.ShapeDtypeStruct((NG, 1), jnp.float32),
    )(out, batch.reshape(N, 1), batch.reshape(1, N),
      lstm_Wih.T, lstm_Whh.T, lstm_bih.reshape(1, 4 * DIM),
      lstm_bhh.reshape(1, 4 * DIM), W1, b1.reshape(1, DIM), W2,
      b2.reshape(1, 1))


def kernel(x, edge_index, edge_attr, batch, W0, b0, We1, be1, We2, be2,
           Wroot, conv_b, gru_Wih, gru_Whh, gru_bih, gru_bhh,
           lstm_Wih, lstm_Whh, lstm_bih, lstm_bhh, W1, b1, W2, b2):
    src = edge_index[0]
    dst3 = edge_index[1].reshape(NW, EPW // CH, CH)
    out = _lin0(x, W0, b0)
    ew_t = _ewt(edge_attr.T, We1, be1, We2, be2)
    zeros32 = jnp.zeros((N, 2 * DIM), jnp.float32)

    countsr = None
    for it in range(3):
        width = 2 * DIM if it == 0 else DIM
        x_src = _gather_sc(out, src)
        msg = _msg(x_src, ew_t, width)
        acc = _scatter_sc(msg, dst3, zeros32[:, :width], width)
        if it == 0:
            countsr = _countsr(acc)
        out = _node(acc, countsr, out, Wroot, conv_b,
                    gru_Wih, gru_Whh, gru_bih, gru_bhh)

    o = _s2s(out, batch, lstm_Wih, lstm_Whh, lstm_bih, lstm_bhh, W1, b1, W2, b2)
    return o.reshape(-1)
